# Initial kernel scaffold; baseline (speedup 1.0000x reference)
import jax
import jax.numpy as jnp
from jax.experimental import pallas as pl


def kernel(nodes, edge_sources, edge_targets, rij, combine_sets, plane_wave, graph_indices, node_counts, params):
    raise NotImplementedError("write your pallas kernel here")


# speedup vs baseline: 3.1538x; 3.1538x over previous
import functools

import jax
import jax.numpy as jnp
from jax import lax
from jax.experimental import pallas as pl
from jax.experimental.pallas import tpu as pltpu
from jax.experimental.pallas import tpu_sc as plsc

_N = 10000
_E = 160000
_F = 128
_K = 64
_G = 64

_NC = 2
_NS = 16
_NW = _NC * _NS
_EPW = _E // _NW
_CH = 40
_NCHUNK = _EPW // _CH
_NCHP = 128
_ROWS_PT = 624
_ROWS_TAIL = _N - _NS * _ROWS_PT

_EB = 2000
_NB = 2000

def _sigmoid(x):
    return 1.0 / (1.0 + jnp.exp(-x))


def _elu(x):
    return jnp.where(x > 0, x, jnp.exp(jnp.minimum(x, 0.0)) - 1.0)


@functools.cache
def _sc_gather_kernel():
    return pl.kernel(
        _sc_gather_body,
        out_type=(jax.ShapeDtypeStruct((_E, _F), jnp.float32),
                  jax.ShapeDtypeStruct((_E, _F), jnp.float32)),
        mesh=plsc.VectorSubcoreMesh(core_axis_name="c", subcore_axis_name="s",
                                    num_cores=_NC, num_subcores=_NS),
        scratch_types=[
            pltpu.VMEM((_NCHP, _CH), jnp.int32),
            pltpu.VMEM((_NCHP, _CH), jnp.int32),
            pltpu.VMEM((2, _CH, _F), jnp.float32),
            pltpu.VMEM((2, _CH, _F), jnp.float32),
            pltpu.SemaphoreType.DMA,
            pltpu.SemaphoreType.DMA,
        ],
    )


def _sc_gather(table, src2, dst2):
    return _sc_gather_kernel()(table, src2, dst2)


def _sc_gather_body(table_h, src_h, dst_h, ni_h, nj_h, sidx, didx, sbuf, dbuf,
                    ssem, dsem):
    c = lax.axis_index("c")
    s = lax.axis_index("s")
    wid = s * _NC + c
    pltpu.sync_copy(src_h.at[pl.ds(wid * _NCHP, _NCHP)], sidx)
    pltpu.sync_copy(dst_h.at[pl.ds(wid * _NCHP, _NCHP)], didx)
    pltpu.async_copy(table_h.at[sidx.at[0]], sbuf.at[0], ssem)
    pltpu.async_copy(table_h.at[didx.at[0]], dbuf.at[0], dsem)
    ebase = wid * _EPW

    def body(j, carry):
        slot = lax.rem(j, 2)

        @pl.when(j + 1 < _NCHUNK)
        def _():
            nxt = lax.rem(j + 1, 2)
            pltpu.async_copy(table_h.at[sidx.at[j + 1]], sbuf.at[nxt], ssem)
            pltpu.async_copy(table_h.at[didx.at[j + 1]], dbuf.at[nxt], dsem)

        pltpu.make_async_copy(table_h.at[sidx.at[j]], sbuf.at[slot], ssem).wait()
        pltpu.sync_copy(sbuf.at[slot], ni_h.at[pl.ds(ebase + j * _CH, _CH)])
        pltpu.make_async_copy(table_h.at[didx.at[j]], dbuf.at[slot], dsem).wait()
        pltpu.sync_copy(dbuf.at[slot], nj_h.at[pl.ds(ebase + j * _CH, _CH)])
        return carry

    lax.fori_loop(0, _NCHUNK, body, 0)


@functools.cache
def _sc_scatter_kernel():
    return pl.kernel(
        _sc_scatter_body,
        out_type=(jax.ShapeDtypeStruct((_N, _F), jnp.float32),
                  jax.ShapeDtypeStruct((_N, _F), jnp.float32)),
        mesh=plsc.VectorSubcoreMesh(core_axis_name="c", subcore_axis_name="s",
                                    num_cores=_NC, num_subcores=_NS),
        scratch_types=[
            pltpu.VMEM((_NCHP, _CH), jnp.int32),
            pltpu.VMEM((2, _CH, _F), jnp.float32),
            pltpu.VMEM_SHARED((_N, _F), jnp.float32),
            pltpu.SemaphoreType.DMA,
        ],
    )


def _sc_scatter(nf, zeros, src2, z):
    return _sc_scatter_kernel()(nf, zeros, src2, z)


def _sc_scatter_body(nf_h, zeros_h, src_h, z_h, p0_h, p1_h, idxb, zbuf, acc,
                     zsem):
    c = lax.axis_index("c")
    s = lax.axis_index("s")
    wid = s * _NC + c
    rbase = s * _ROWS_PT

    @pl.when(c == 0)
    def _():
        pltpu.sync_copy(nf_h.at[pl.ds(rbase, _ROWS_PT)],
                        acc.at[pl.ds(rbase, _ROWS_PT)])

        @pl.when(s == _NS - 1)
        def _():
            pltpu.sync_copy(nf_h.at[pl.ds(_NS * _ROWS_PT, _ROWS_TAIL)],
                            acc.at[pl.ds(_NS * _ROWS_PT, _ROWS_TAIL)])

    @pl.when(c == 1)
    def _():
        pltpu.sync_copy(zeros_h.at[pl.ds(rbase, _ROWS_PT)],
                        acc.at[pl.ds(rbase, _ROWS_PT)])

        @pl.when(s == _NS - 1)
        def _():
            pltpu.sync_copy(zeros_h.at[pl.ds(_NS * _ROWS_PT, _ROWS_TAIL)],
                            acc.at[pl.ds(_NS * _ROWS_PT, _ROWS_TAIL)])

    pltpu.sync_copy(src_h.at[pl.ds(wid * _NCHP, _NCHP)], idxb)
    ebase = wid * _EPW
    pltpu.async_copy(z_h.at[pl.ds(ebase, _CH)], zbuf.at[0], zsem)
    plsc.subcore_barrier()

    def body(j, carry):
        slot = lax.rem(j, 2)

        @pl.when(j + 1 < _NCHUNK)
        def _():
            pltpu.async_copy(z_h.at[pl.ds(ebase + (j + 1) * _CH, _CH)],
                             zbuf.at[lax.rem(j + 1, 2)], zsem)

        pltpu.make_async_copy(z_h.at[pl.ds(ebase + j * _CH, _CH)],
                              zbuf.at[slot], zsem).wait()
        pltpu.sync_copy(zbuf.at[slot], acc.at[idxb.at[j]], add=True)
        return carry

    lax.fori_loop(0, _NCHUNK, body, 0)
    plsc.subcore_barrier()

    @pl.when(c == 0)
    def _():
        pltpu.sync_copy(acc.at[pl.ds(rbase, _ROWS_PT)],
                        p0_h.at[pl.ds(rbase, _ROWS_PT)])

        @pl.when(s == _NS - 1)
        def _():
            pltpu.sync_copy(acc.at[pl.ds(_NS * _ROWS_PT, _ROWS_TAIL)],
                            p0_h.at[pl.ds(_NS * _ROWS_PT, _ROWS_TAIL)])

    @pl.when(c == 1)
    def _():
        pltpu.sync_copy(acc.at[pl.ds(rbase, _ROWS_PT)],
                        p1_h.at[pl.ds(rbase, _ROWS_PT)])

        @pl.when(s == _NS - 1)
        def _():
            pltpu.sync_copy(acc.at[pl.ds(_NS * _ROWS_PT, _ROWS_TAIL)],
                            p1_h.at[pl.ds(_NS * _ROWS_PT, _ROWS_TAIL)])


def _embed_body(x_ref, w_ref, o_ref):
    o_ref[...] = _sigmoid(
        jnp.dot(x_ref[...], w_ref[...], preferred_element_type=jnp.float32))


def _tc_embed(nodes, w_emb):
    return pl.pallas_call(
        _embed_body,
        grid=(_N // _NB,),
        in_specs=[pl.BlockSpec((_NB, _F), lambda i: (i, 0)),
                  pl.BlockSpec((_F, _F), lambda i: (0, 0))],
        out_specs=pl.BlockSpec((_NB, _F), lambda i: (i, 0)),
        out_shape=jax.ShapeDtypeStruct((_N, _F), jnp.float32),
    )(nodes, w_emb)


def _edge_body(ni_ref, nj_ref, r_ref, cs_ref, pw_ref, w1_ref, w2_ref, w3_ref,
               w1v_ref, w2vg_ref, w2v_ref, z_ref):
    ni = ni_ref[...]
    nj = nj_ref[...]
    r = r_ref[...]
    rinv = 1.0 / r
    mask = (r < 8.0).astype(jnp.float32)
    p1 = jnp.dot(ni, w1_ref[...], preferred_element_type=jnp.float32)
    p2 = jnp.dot(nj, w2_ref[...], preferred_element_type=jnp.float32)
    p3 = jnp.dot(ni - nj, w3_ref[...], preferred_element_type=jnp.float32)
    pre = p1 + p2 + rinv * p3
    eg = _sigmoid(pre[:, :_F])
    em = _elu(pre[:, _F:])
    pw = pw_ref[...]
    gate = _sigmoid(jnp.dot(pw, w2vg_ref[...],
                            preferred_element_type=jnp.float32))
    z12 = (jnp.dot(cs_ref[...], w1v_ref[...],
                   preferred_element_type=jnp.float32)
           + jnp.dot(pw * gate, w2v_ref[...],
                     preferred_element_type=jnp.float32))
    z_ref[...] = eg * em * z12 * mask


def _tc_edge(ni, nj, r2, cs, pw, w1, w2, w3, w1v, w2vg, w2v):
    full = lambda shape: pl.BlockSpec(shape, lambda i: (0, 0))
    return pl.pallas_call(
        _edge_body,
        grid=(_E // _EB,),
        in_specs=[
            pl.BlockSpec((_EB, _F), lambda i: (i, 0)),
            pl.BlockSpec((_EB, _F), lambda i: (i, 0)),
            pl.BlockSpec((_EB, 1), lambda i: (i, 0)),
            pl.BlockSpec((_EB, _K), lambda i: (i, 0)),
            pl.BlockSpec((_EB, _K), lambda i: (i, 0)),
            full((_F, 2 * _F)),
            full((_F, 2 * _F)),
            full((_F, 2 * _F)),
            full((_K, _F)),
            full((_K, _K)),
            full((_K, _F)),
        ],
        out_specs=pl.BlockSpec((_EB, _F), lambda i: (i, 0)),
        out_shape=jax.ShapeDtypeStruct((_E, _F), jnp.float32),
        compiler_params=pltpu.CompilerParams(
            dimension_semantics=("parallel",)),
    )(ni, nj, r2, cs, pw, w1, w2, w3, w1v, w2vg, w2v)


def _node_body(p0_ref, p1_ref, gi_ref, wpsi_ref, wp1_ref, wp2_ref,
               nf_ref, gf_ref):
    i = pl.program_id(0)
    nf = _elu(jnp.dot(p0_ref[...] + p1_ref[...], wpsi_ref[...],
                      preferred_element_type=jnp.float32))
    nf_ref[...] = nf
    pz = (_elu(jnp.dot(nf, wp1_ref[...], preferred_element_type=jnp.float32))
          * jnp.dot(nf, wp2_ref[...], preferred_element_type=jnp.float32))
    gids = lax.broadcasted_iota(jnp.int32, (_NB, _G), 1)
    oh = (gi_ref[...] == gids).astype(jnp.float32)
    contrib = lax.dot_general(oh, pz, (((0,), (0,)), ((), ())))

    @pl.when(i == 0)
    def _():
        gf_ref[...] = contrib

    @pl.when(i > 0)
    def _():
        gf_ref[...] = gf_ref[...] + contrib


def _tc_node(p0, p1, gi2, wpsi, wp1, wp2):
    return pl.pallas_call(
        _node_body,
        grid=(_N // _NB,),
        in_specs=[
            pl.BlockSpec((_NB, _F), lambda i: (i, 0)),
            pl.BlockSpec((_NB, _F), lambda i: (i, 0)),
            pl.BlockSpec((_NB, 1), lambda i: (i, 0)),
            pl.BlockSpec((_F, _F), lambda i: (0, 0)),
            pl.BlockSpec((_F, _F), lambda i: (0, 0)),
            pl.BlockSpec((_F, _F), lambda i: (0, 0)),
        ],
        out_specs=(pl.BlockSpec((_NB, _F), lambda i: (i, 0)),
                   pl.BlockSpec((_G, _F), lambda i: (0, 0))),
        out_shape=(jax.ShapeDtypeStruct((_N, _F), jnp.float32),
                   jax.ShapeDtypeStruct((_G, _F), jnp.float32)),
        compiler_params=pltpu.CompilerParams(
            dimension_semantics=("arbitrary",)),
    )(p0, p1, gi2, wpsi, wp1, wp2)


def _head_body(gf_ref, w1_ref, w2_ref, w3_ref, y_ref):
    y = _elu(jnp.dot(gf_ref[...], w1_ref[...],
                     preferred_element_type=jnp.float32))
    y = _elu(jnp.dot(y, w2_ref[...], preferred_element_type=jnp.float32))
    y_ref[...] = jnp.dot(y, w3_ref[...], preferred_element_type=jnp.float32)


def _tc_head(gf, w1, w2, w3):
    return pl.pallas_call(
        _head_body,
        out_shape=jax.ShapeDtypeStruct((_G, 1), jnp.float32),
    )(gf, w1, w2, w3)


def kernel(nodes, edge_sources, edge_targets, rij, combine_sets, plane_wave,
           graph_indices, node_counts, params):
    del node_counts
    nf = _tc_embed(nodes, params["W_emb"])
    pad = ((0, 0), (0, _NCHP - _NCHUNK), (0, 0))
    src2 = jnp.pad(edge_sources.reshape(_NW, _NCHUNK, _CH),
                   pad).reshape(_NW * _NCHP, _CH)
    dst2 = jnp.pad(edge_targets.reshape(_NW, _NCHUNK, _CH),
                   pad).reshape(_NW * _NCHP, _CH)
    r2 = rij.reshape(_E, 1)
    gi2 = graph_indices.reshape(_N, 1)
    zeros = jnp.zeros((_N, _F), jnp.float32)
    gf = None
    for blk in params["blocks"]:
        wg, wm = blk["W_gate"], blk["W_mlp"]
        w1 = jnp.concatenate([wg[:_F], wm[:_F]], axis=1)
        w2 = jnp.concatenate([wg[_F:2 * _F], wm[_F:2 * _F]], axis=1)
        w3 = jnp.concatenate([wg[2 * _F:], wm[2 * _F:]], axis=1)
        ni, nj = _sc_gather(nf, src2, dst2)
        z = _tc_edge(ni, nj, r2, combine_sets, plane_wave, w1, w2, w3,
                     blk["W_1v"], blk["W_2vg"], blk["W_2v"])
        p0, p1 = _sc_scatter(nf, zeros, src2, z)
        nf, gfb = _tc_node(p0, p1, gi2, blk["W_psi"], blk["W_p1"], blk["W_p2"])
        gf = gfb if gf is None else gf + gfb
    return _tc_head(gf, params["W_lr1"], params["W_lr2"], params["W_lr3"])


# speedup vs baseline: 3.4025x; 1.0789x over previous
import functools

import jax
import jax.numpy as jnp
from jax import lax
from jax.experimental import pallas as pl
from jax.experimental.pallas import tpu as pltpu
from jax.experimental.pallas import tpu_sc as plsc

_N = 10000
_E = 160000
_F = 128
_K = 64
_G = 64

_NC = 2
_NS = 16
_NW = _NC * _NS
_CH = 128
_NCH0 = 39
_NCH1 = 41
_EPW = _NCH0 * _CH
_NCHP = 48
_ROWS_PT = 624
_ROWS_TAIL = _N - _NS * _ROWS_PT

_EB = 2000
_NB = 2000

def _sigmoid(x):
    return 1.0 / (1.0 + jnp.exp(-x))


def _elu(x):
    return jnp.where(x > 0, x, jnp.exp(jnp.minimum(x, 0.0)) - 1.0)


@functools.cache
def _sc_gather_kernel():
    return pl.kernel(
        _sc_gather_body,
        out_type=(jax.ShapeDtypeStruct((_E, _F), jnp.float32),
                  jax.ShapeDtypeStruct((_E, _F), jnp.float32)),
        mesh=plsc.VectorSubcoreMesh(core_axis_name="c", subcore_axis_name="s",
                                    num_cores=_NC, num_subcores=_NS),
        scratch_types=[
            pltpu.VMEM((_NCHP, _CH), jnp.int32),
            pltpu.VMEM((_NCHP, _CH), jnp.int32),
            pltpu.VMEM((2, _CH, _F), jnp.float32),
            pltpu.VMEM((2, _CH, _F), jnp.float32),
            pltpu.SemaphoreType.DMA,
            pltpu.SemaphoreType.DMA,
        ],
    )


def _sc_gather(table, src2, dst2):
    return _sc_gather_kernel()(table, src2, dst2)


def _sc_gather_body(table_h, src_h, dst_h, ni_h, nj_h, sidx, didx, sbuf, dbuf,
                    ssem, dsem):
    c = lax.axis_index("c")
    s = lax.axis_index("s")
    wid = s * _NC + c
    nch = jnp.where(wid == _NW - 1, _NCH1, _NCH0)
    pltpu.sync_copy(src_h.at[pl.ds(wid * _NCHP, _NCHP)], sidx)
    pltpu.sync_copy(dst_h.at[pl.ds(wid * _NCHP, _NCHP)], didx)
    pltpu.async_copy(table_h.at[sidx.at[0]], sbuf.at[0], ssem)
    pltpu.async_copy(table_h.at[didx.at[0]], dbuf.at[0], dsem)
    ebase = wid * _EPW

    def body(j, carry):
        slot = lax.rem(j, 2)

        @pl.when(j + 1 < nch)
        def _():
            nxt = lax.rem(j + 1, 2)
            pltpu.async_copy(table_h.at[sidx.at[j + 1]], sbuf.at[nxt], ssem)
            pltpu.async_copy(table_h.at[didx.at[j + 1]], dbuf.at[nxt], dsem)

        pltpu.make_async_copy(table_h.at[sidx.at[j]], sbuf.at[slot], ssem).wait()
        pltpu.sync_copy(sbuf.at[slot], ni_h.at[pl.ds(ebase + j * _CH, _CH)])
        pltpu.make_async_copy(table_h.at[didx.at[j]], dbuf.at[slot], dsem).wait()
        pltpu.sync_copy(dbuf.at[slot], nj_h.at[pl.ds(ebase + j * _CH, _CH)])
        return carry

    lax.fori_loop(0, nch, body, 0)


@functools.cache
def _sc_scatter_kernel():
    return pl.kernel(
        _sc_scatter_body,
        out_type=(jax.ShapeDtypeStruct((_N, _F), jnp.float32),
                  jax.ShapeDtypeStruct((_N, _F), jnp.float32)),
        mesh=plsc.VectorSubcoreMesh(core_axis_name="c", subcore_axis_name="s",
                                    num_cores=_NC, num_subcores=_NS),
        scratch_types=[
            pltpu.VMEM((_NCHP, _CH), jnp.int32),
            pltpu.VMEM((2, _CH, _F), jnp.float32),
            pltpu.VMEM_SHARED((_N, _F), jnp.float32),
            pltpu.SemaphoreType.DMA,
        ],
    )


def _sc_scatter(nf, zeros, src2, z):
    return _sc_scatter_kernel()(nf, zeros, src2, z)


def _sc_scatter_body(nf_h, zeros_h, src_h, z_h, p0_h, p1_h, idxb, zbuf, acc,
                     zsem):
    c = lax.axis_index("c")
    s = lax.axis_index("s")
    wid = s * _NC + c
    rbase = s * _ROWS_PT

    @pl.when(c == 0)
    def _():
        pltpu.sync_copy(nf_h.at[pl.ds(rbase, _ROWS_PT)],
                        acc.at[pl.ds(rbase, _ROWS_PT)])

        @pl.when(s == _NS - 1)
        def _():
            pltpu.sync_copy(nf_h.at[pl.ds(_NS * _ROWS_PT, _ROWS_TAIL)],
                            acc.at[pl.ds(_NS * _ROWS_PT, _ROWS_TAIL)])

    @pl.when(c == 1)
    def _():
        pltpu.sync_copy(zeros_h.at[pl.ds(rbase, _ROWS_PT)],
                        acc.at[pl.ds(rbase, _ROWS_PT)])

        @pl.when(s == _NS - 1)
        def _():
            pltpu.sync_copy(zeros_h.at[pl.ds(_NS * _ROWS_PT, _ROWS_TAIL)],
                            acc.at[pl.ds(_NS * _ROWS_PT, _ROWS_TAIL)])

    pltpu.sync_copy(src_h.at[pl.ds(wid * _NCHP, _NCHP)], idxb)
    nch = jnp.where(wid == _NW - 1, _NCH1, _NCH0)
    ebase = wid * _EPW
    pltpu.async_copy(z_h.at[pl.ds(ebase, _CH)], zbuf.at[0], zsem)
    plsc.subcore_barrier()

    def body(j, carry):
        slot = lax.rem(j, 2)

        @pl.when(j + 1 < nch)
        def _():
            pltpu.async_copy(z_h.at[pl.ds(ebase + (j + 1) * _CH, _CH)],
                             zbuf.at[lax.rem(j + 1, 2)], zsem)

        pltpu.make_async_copy(z_h.at[pl.ds(ebase + j * _CH, _CH)],
                              zbuf.at[slot], zsem).wait()
        pltpu.sync_copy(zbuf.at[slot], acc.at[idxb.at[j]], add=True)
        return carry

    lax.fori_loop(0, nch, body, 0)
    plsc.subcore_barrier()

    @pl.when(c == 0)
    def _():
        pltpu.sync_copy(acc.at[pl.ds(rbase, _ROWS_PT)],
                        p0_h.at[pl.ds(rbase, _ROWS_PT)])

        @pl.when(s == _NS - 1)
        def _():
            pltpu.sync_copy(acc.at[pl.ds(_NS * _ROWS_PT, _ROWS_TAIL)],
                            p0_h.at[pl.ds(_NS * _ROWS_PT, _ROWS_TAIL)])

    @pl.when(c == 1)
    def _():
        pltpu.sync_copy(acc.at[pl.ds(rbase, _ROWS_PT)],
                        p1_h.at[pl.ds(rbase, _ROWS_PT)])

        @pl.when(s == _NS - 1)
        def _():
            pltpu.sync_copy(acc.at[pl.ds(_NS * _ROWS_PT, _ROWS_TAIL)],
                            p1_h.at[pl.ds(_NS * _ROWS_PT, _ROWS_TAIL)])


def _embed_body(x_ref, w_ref, o_ref):
    o_ref[...] = _sigmoid(
        jnp.dot(x_ref[...], w_ref[...], preferred_element_type=jnp.float32))


def _tc_embed(nodes, w_emb):
    return pl.pallas_call(
        _embed_body,
        grid=(_N // _NB,),
        in_specs=[pl.BlockSpec((_NB, _F), lambda i: (i, 0)),
                  pl.BlockSpec((_F, _F), lambda i: (0, 0))],
        out_specs=pl.BlockSpec((_NB, _F), lambda i: (i, 0)),
        out_shape=jax.ShapeDtypeStruct((_N, _F), jnp.float32),
    )(nodes, w_emb)


def _edge_body(ni_ref, nj_ref, r_ref, cs_ref, pw_ref, w1_ref, w2_ref, w3_ref,
               w1v_ref, w2vg_ref, w2v_ref, z_ref):
    ni = ni_ref[...]
    nj = nj_ref[...]
    r = r_ref[...]
    rinv = 1.0 / r
    mask = (r < 8.0).astype(jnp.float32)
    p1 = jnp.dot(ni, w1_ref[...], preferred_element_type=jnp.float32)
    p2 = jnp.dot(nj, w2_ref[...], preferred_element_type=jnp.float32)
    p3 = jnp.dot(ni - nj, w3_ref[...], preferred_element_type=jnp.float32)
    pre = p1 + p2 + rinv * p3
    eg = _sigmoid(pre[:, :_F])
    em = _elu(pre[:, _F:])
    pw = pw_ref[...]
    gate = _sigmoid(jnp.dot(pw, w2vg_ref[...],
                            preferred_element_type=jnp.float32))
    z12 = (jnp.dot(cs_ref[...], w1v_ref[...],
                   preferred_element_type=jnp.float32)
           + jnp.dot(pw * gate, w2v_ref[...],
                     preferred_element_type=jnp.float32))
    z_ref[...] = eg * em * z12 * mask


def _tc_edge(ni, nj, r2, cs, pw, w1, w2, w3, w1v, w2vg, w2v):
    full = lambda shape: pl.BlockSpec(shape, lambda i: (0, 0))
    return pl.pallas_call(
        _edge_body,
        grid=(_E // _EB,),
        in_specs=[
            pl.BlockSpec((_EB, _F), lambda i: (i, 0)),
            pl.BlockSpec((_EB, _F), lambda i: (i, 0)),
            pl.BlockSpec((_EB, 1), lambda i: (i, 0)),
            pl.BlockSpec((_EB, _K), lambda i: (i, 0)),
            pl.BlockSpec((_EB, _K), lambda i: (i, 0)),
            full((_F, 2 * _F)),
            full((_F, 2 * _F)),
            full((_F, 2 * _F)),
            full((_K, _F)),
            full((_K, _K)),
            full((_K, _F)),
        ],
        out_specs=pl.BlockSpec((_EB, _F), lambda i: (i, 0)),
        out_shape=jax.ShapeDtypeStruct((_E, _F), jnp.float32),
        compiler_params=pltpu.CompilerParams(
            dimension_semantics=("parallel",)),
    )(ni, nj, r2, cs, pw, w1, w2, w3, w1v, w2vg, w2v)


def _node_body(p0_ref, p1_ref, gi_ref, wpsi_ref, wp1_ref, wp2_ref,
               nf_ref, gf_ref):
    i = pl.program_id(0)
    nf = _elu(jnp.dot(p0_ref[...] + p1_ref[...], wpsi_ref[...],
                      preferred_element_type=jnp.float32))
    nf_ref[...] = nf
    pz = (_elu(jnp.dot(nf, wp1_ref[...], preferred_element_type=jnp.float32))
          * jnp.dot(nf, wp2_ref[...], preferred_element_type=jnp.float32))
    gids = lax.broadcasted_iota(jnp.int32, (_NB, _G), 1)
    oh = (gi_ref[...] == gids).astype(jnp.float32)
    contrib = lax.dot_general(oh, pz, (((0,), (0,)), ((), ())))

    @pl.when(i == 0)
    def _():
        gf_ref[...] = contrib

    @pl.when(i > 0)
    def _():
        gf_ref[...] = gf_ref[...] + contrib


def _tc_node(p0, p1, gi2, wpsi, wp1, wp2):
    return pl.pallas_call(
        _node_body,
        grid=(_N // _NB,),
        in_specs=[
            pl.BlockSpec((_NB, _F), lambda i: (i, 0)),
            pl.BlockSpec((_NB, _F), lambda i: (i, 0)),
            pl.BlockSpec((_NB, 1), lambda i: (i, 0)),
            pl.BlockSpec((_F, _F), lambda i: (0, 0)),
            pl.BlockSpec((_F, _F), lambda i: (0, 0)),
            pl.BlockSpec((_F, _F), lambda i: (0, 0)),
        ],
        out_specs=(pl.BlockSpec((_NB, _F), lambda i: (i, 0)),
                   pl.BlockSpec((_G, _F), lambda i: (0, 0))),
        out_shape=(jax.ShapeDtypeStruct((_N, _F), jnp.float32),
                   jax.ShapeDtypeStruct((_G, _F), jnp.float32)),
        compiler_params=pltpu.CompilerParams(
            dimension_semantics=("arbitrary",)),
    )(p0, p1, gi2, wpsi, wp1, wp2)


def _head_body(gf_ref, w1_ref, w2_ref, w3_ref, y_ref):
    y = _elu(jnp.dot(gf_ref[...], w1_ref[...],
                     preferred_element_type=jnp.float32))
    y = _elu(jnp.dot(y, w2_ref[...], preferred_element_type=jnp.float32))
    y_ref[...] = jnp.dot(y, w3_ref[...], preferred_element_type=jnp.float32)


def _tc_head(gf, w1, w2, w3):
    return pl.pallas_call(
        _head_body,
        out_shape=jax.ShapeDtypeStruct((_G, 1), jnp.float32),
    )(gf, w1, w2, w3)


def _pad_idx(flat):
    pieces = []
    for w in range(_NW):
        n = _NCH1 if w == _NW - 1 else _NCH0
        blk = flat[w * _EPW: w * _EPW + n * _CH].reshape(n, _CH)
        pieces.append(jnp.pad(blk, ((0, _NCHP - n), (0, 0))))
    return jnp.concatenate(pieces, axis=0)


def kernel(nodes, edge_sources, edge_targets, rij, combine_sets, plane_wave,
           graph_indices, node_counts, params):
    del node_counts
    nf = _tc_embed(nodes, params["W_emb"])
    src2 = _pad_idx(edge_sources)
    dst2 = _pad_idx(edge_targets)
    r2 = rij.reshape(_E, 1)
    gi2 = graph_indices.reshape(_N, 1)
    zeros = jnp.zeros((_N, _F), jnp.float32)
    gf = None
    for blk in params["blocks"]:
        wg, wm = blk["W_gate"], blk["W_mlp"]
        w1 = jnp.concatenate([wg[:_F], wm[:_F]], axis=1)
        w2 = jnp.concatenate([wg[_F:2 * _F], wm[_F:2 * _F]], axis=1)
        w3 = jnp.concatenate([wg[2 * _F:], wm[2 * _F:]], axis=1)
        ni, nj = _sc_gather(nf, src2, dst2)
        z = _tc_edge(ni, nj, r2, combine_sets, plane_wave, w1, w2, w3,
                     blk["W_1v"], blk["W_2vg"], blk["W_2v"])
        p0, p1 = _sc_scatter(nf, zeros, src2, z)
        nf, gfb = _tc_node(p0, p1, gi2, blk["W_psi"], blk["W_p1"], blk["W_p2"])
        gf = gfb if gf is None else gf + gfb
    return _tc_head(gf, params["W_lr1"], params["W_lr2"], params["W_lr3"])
